# baseline (speedup 1.0000x reference)
import jax
import jax.numpy as jnp
from jax import lax
from jax.experimental import pallas as pl
from jax.experimental.pallas import tpu as pltpu
from jax.experimental.pallas import tpu_sc as plsc

E = 8
K = 2
D = 1024
F = 2048
T = 2048
CAP = int(T * K / E * 1.25)
CAPP = CAP + 1
TAU = 0.8

_INFO = plsc.get_sparse_core_info()
_NC, _NS = _INFO.num_cores, _INFO.num_subcores
_NW = _NC * _NS

_BT = 256
_DP = D // 2
_NP = T * K
_NB = 32
_PB = _NP // _NB
_NSLOT = E * CAPP


def _route_body(x_ref, cov_ref, mu_ref, eidx_ref, gate_ref):
    x = x_ref[...]
    cov = cov_ref[...]
    mu = mu_ref[...]
    xc = jnp.dot(x, cov, preferred_element_type=jnp.float32)
    x_term = jnp.sum(xc * x, axis=1, keepdims=True)
    cross = lax.dot_general(xc, mu, (((1,), (1,)), ((), ())),
                            preferred_element_type=jnp.float32)
    muc = jnp.dot(mu, cov, preferred_element_type=jnp.float32)
    mu_term = jnp.sum(muc * mu, axis=1)
    dist = x_term - 2.0 * cross + mu_term[None, :]
    logits = -dist * (1.0 / (TAU * (float(D) ** 0.5)))
    m = jnp.max(logits, axis=1, keepdims=True)
    ex = jnp.exp(logits - m)
    p = ex / jnp.sum(ex, axis=1, keepdims=True)
    iota = lax.broadcasted_iota(jnp.int32, (_BT, E), 1)
    p1 = jnp.max(p, axis=1, keepdims=True)
    i1 = jnp.min(jnp.where(p == p1, iota, E), axis=1, keepdims=True)
    pm = jnp.where(iota == i1, -1.0, p)
    p2 = jnp.max(pm, axis=1, keepdims=True)
    i2 = jnp.min(jnp.where(pm == p2, iota, E), axis=1, keepdims=True)
    den = p1 + p2
    eidx_ref[...] = jnp.concatenate([i1, i2], axis=1)
    gate_ref[...] = jnp.concatenate([p1 / den, p2 / den], axis=1)


def _route(x, mu, cov_inv):
    return pl.pallas_call(
        _route_body,
        grid=(T // _BT,),
        in_specs=[
            pl.BlockSpec((_BT, D), lambda i: (i, 0)),
            pl.BlockSpec((D, D), lambda i: (0, 0)),
            pl.BlockSpec((E, D), lambda i: (0, 0)),
        ],
        out_specs=[
            pl.BlockSpec((_BT, K), lambda i: (i, 0)),
            pl.BlockSpec((_BT, K), lambda i: (i, 0)),
        ],
        out_shape=[
            jax.ShapeDtypeStruct((T, K), jnp.int32),
            jax.ShapeDtypeStruct((T, K), jnp.float32),
        ],
    )(x, cov_inv, mu)


def _disp_body(e_ref, g_ref, slot_ref, scale_ref):
    ef = e_ref[...]
    eiota = lax.broadcasted_iota(jnp.int32, (_NB, E, _PB), 1)
    oh = (ef[:, None, :] == eiota).astype(jnp.float32)
    ii = lax.broadcasted_iota(jnp.int32, (_PB, _PB), 0)
    jj = lax.broadcasted_iota(jnp.int32, (_PB, _PB), 1)
    tri = (jj <= ii).astype(jnp.float32)
    within = lax.dot_general(oh, tri, (((2,), (1,)), ((), ())),
                             preferred_element_type=jnp.float32)
    totals = jnp.sum(oh, axis=2)
    bi = lax.broadcasted_iota(jnp.int32, (_NB, _NB), 0)
    bj = lax.broadcasted_iota(jnp.int32, (_NB, _NB), 1)
    triex = (bj < bi).astype(jnp.float32)
    offs = lax.dot_general(triex, totals, (((1,), (0,)), ((), ())))
    pos = jnp.sum(oh * (within + offs[:, :, None]), axis=1) - 1.0
    posi = pos.astype(jnp.int32)
    keep = posi < CAP
    posc = jnp.minimum(posi, CAP - 1)
    slot_ref[...] = jnp.where(keep, ef * CAPP + posc, ef * CAPP + CAP)
    scale_ref[...] = jnp.where(keep, g_ref[...], 0.0)


def _dispatch(e2d, g2d):
    return pl.pallas_call(
        _disp_body,
        out_shape=[
            jax.ShapeDtypeStruct((_NB, _PB), jnp.int32),
            jax.ShapeDtypeStruct((_NB, _PB), jnp.float32),
        ],
    )(e2d, g2d)


_ST = T // _NW


def _scatterx_body(x_hbm, s0_hbm, s1_hbm, xg_hbm, slab_v, i0_v, i1_v, sem):
    wid = lax.axis_index("s") * _NC + lax.axis_index("c")
    tbase = wid * _ST
    ca = pltpu.async_copy(x_hbm.at[pl.ds(tbase, _ST)], slab_v, sem)
    cb = pltpu.async_copy(s0_hbm.at[pl.ds(tbase, _ST)], i0_v, sem)
    cc = pltpu.async_copy(s1_hbm.at[pl.ds(tbase, _ST)], i1_v, sem)
    ca.wait()
    cb.wait()
    cc.wait()
    c0 = pltpu.async_copy(slab_v, xg_hbm.at[i0_v], sem)
    c1 = pltpu.async_copy(slab_v, xg_hbm.at[i1_v], sem)
    c0.wait()
    c1.wait()


def _scatter_x(x, slot0, slot1):
    mesh = plsc.VectorSubcoreMesh(core_axis_name="c", subcore_axis_name="s")
    return pl.kernel(
        _scatterx_body,
        mesh=mesh,
        out_type=jax.ShapeDtypeStruct((_NSLOT, _DP), jnp.float32),
        scratch_types=[
            pltpu.VMEM((_ST, _DP), jnp.float32),
            pltpu.VMEM((_ST,), jnp.int32),
            pltpu.VMEM((_ST,), jnp.int32),
            pltpu.SemaphoreType.DMA,
        ],
    )(x, slot0, slot1)


def _ffn_body(xg_ref, w1_ref, b1_ref, w2_ref, b2_ref, out_ref):
    xg = xg_ref[0].astype(jnp.float32)
    h = jnp.dot(xg, w1_ref[0], preferred_element_type=jnp.float32)
    h = jnp.maximum(h + b1_ref[0], 0.0)
    o = jnp.dot(h, w2_ref[0], preferred_element_type=jnp.float32)
    out_ref[0] = (o + b2_ref[0]).astype(jnp.bfloat16)


def _ffn(xg, W1, b1, W2, b2):
    return pl.pallas_call(
        _ffn_body,
        grid=(E,),
        in_specs=[
            pl.BlockSpec((1, CAPP, D), lambda e: (e, 0, 0)),
            pl.BlockSpec((1, D, F), lambda e: (e, 0, 0)),
            pl.BlockSpec((1, 1, F), lambda e: (e, 0, 0)),
            pl.BlockSpec((1, F, D), lambda e: (e, 0, 0)),
            pl.BlockSpec((1, 1, D), lambda e: (e, 0, 0)),
        ],
        out_specs=pl.BlockSpec((1, CAPP, D), lambda e: (e, 0, 0)),
        out_shape=jax.ShapeDtypeStruct((E, CAPP, D), jnp.bfloat16),
    )(xg, W1, b1.reshape(E, 1, F), W2, b2.reshape(E, 1, D))


_CR = _NP // _NW
_CCH = 32


def _gathero_body(oute_hbm, f0_hbm, f1_hbm, g0_hbm, g1_hbm,
                  b0, b1, i0, i1, sem, isem):
    wid = lax.axis_index("s") * _NC + lax.axis_index("c")
    base = wid * _ST
    ca = pltpu.async_copy(f0_hbm.at[pl.ds(base, _ST)], i0, isem)
    cb = pltpu.async_copy(f1_hbm.at[pl.ds(base, _ST)], i1, isem)
    ca.wait()
    g0c = pltpu.async_copy(oute_hbm.at[i0], b0, sem)
    cb.wait()
    g1c = pltpu.async_copy(oute_hbm.at[i1], b1, sem)
    g0c.wait()
    pltpu.sync_copy(b0, g0_hbm.at[pl.ds(base, _ST)])
    g1c.wait()
    pltpu.sync_copy(b1, g1_hbm.at[pl.ds(base, _ST)])


def _gather_o(oute, fsrc0, fsrc1):
    mesh = plsc.VectorSubcoreMesh(core_axis_name="c", subcore_axis_name="s")
    return pl.kernel(
        _gathero_body,
        mesh=mesh,
        out_type=[
            jax.ShapeDtypeStruct((T, _DP), jnp.float32),
            jax.ShapeDtypeStruct((T, _DP), jnp.float32),
        ],
        scratch_types=[
            pltpu.VMEM((_ST, _DP), jnp.float32),
            pltpu.VMEM((_ST, _DP), jnp.float32),
            pltpu.VMEM((_ST,), jnp.int32),
            pltpu.VMEM((_ST,), jnp.int32),
            pltpu.SemaphoreType.DMA,
            pltpu.SemaphoreType.DMA,
        ],
    )(oute, fsrc0, fsrc1)


def _combine_body(g0_ref, g1_ref, s_ref, y_ref):
    u0 = g0_ref[...].astype(jnp.float32)
    u1 = g1_ref[...].astype(jnp.float32)
    s = s_ref[...]
    y_ref[...] = u0 * s[:, 0:1] + u1 * s[:, 1:2]


def _combine(g0, g1, scale):
    return pl.pallas_call(
        _combine_body,
        grid=(T // _BT,),
        in_specs=[
            pl.BlockSpec((_BT, D), lambda i: (i, 0)),
            pl.BlockSpec((_BT, D), lambda i: (i, 0)),
            pl.BlockSpec((_BT, K), lambda i: (i, 0)),
        ],
        out_specs=pl.BlockSpec((_BT, D), lambda i: (i, 0)),
        out_shape=jax.ShapeDtypeStruct((T, D), jnp.float32),
    )(g0, g1, scale)


def kernel(x, mu, cov_inv, W1, b1, W2, b2):
    eidx, gates = _route(x, mu, cov_inv)
    slot2d, scale2d = _dispatch(eidx.reshape(_NB, _PB), gates.reshape(_NB, _PB))
    slotk = slot2d.reshape(T, K)
    xp = lax.bitcast_convert_type(
        x.astype(jnp.bfloat16).reshape(T, _DP, 2), jnp.float32)
    xg = _scatter_x(xp, slotk[:, 0], slotk[:, 1])
    xgu = lax.bitcast_convert_type(xg, jnp.bfloat16).reshape(E, CAPP, D)
    oute = _ffn(xgu, W1, b1, W2, b2)
    outp = lax.bitcast_convert_type(
        oute.reshape(_NSLOT, _DP, 2), jnp.float32)
    g0, g1 = _gather_o(outp, slotk[:, 0], slotk[:, 1])
    u0 = lax.bitcast_convert_type(g0, jnp.bfloat16).reshape(T, D)
    u1 = lax.bitcast_convert_type(g1, jnp.bfloat16).reshape(T, D)
    y = _combine(u0, u1, scale2d.reshape(T, K))
    return y


# speedup vs baseline: 2.2624x; 2.2624x over previous
import jax
import jax.numpy as jnp
from jax import lax
from jax.experimental import pallas as pl
from jax.experimental.pallas import tpu as pltpu
from jax.experimental.pallas import tpu_sc as plsc

E = 8
K = 2
D = 1024
F = 2048
T = 2048
CAP = int(T * K / E * 1.25)
CAPP = CAP + 1
TAU = 0.8

_INFO = plsc.get_sparse_core_info()
_NC, _NS = _INFO.num_cores, _INFO.num_subcores
_NW = _NC * _NS

_BT = 256
_DP = D // 2
_NP = T * K
_NB = 32
_PB = _NP // _NB
_NSLOT = E * CAPP


def _route_body(x_ref, cov_ref, mu_ref, eidx_ref, gate_ref):
    x = x_ref[...]
    cov = cov_ref[...]
    mu = mu_ref[...]
    xc = jnp.dot(x, cov, preferred_element_type=jnp.float32)
    x_term = jnp.sum(xc * x, axis=1, keepdims=True)
    cross = lax.dot_general(xc, mu, (((1,), (1,)), ((), ())),
                            preferred_element_type=jnp.float32)
    muc = jnp.dot(mu, cov, preferred_element_type=jnp.float32)
    mu_term = jnp.sum(muc * mu, axis=1)
    dist = x_term - 2.0 * cross + mu_term[None, :]
    logits = -dist * (1.0 / (TAU * (float(D) ** 0.5)))
    m = jnp.max(logits, axis=1, keepdims=True)
    ex = jnp.exp(logits - m)
    p = ex / jnp.sum(ex, axis=1, keepdims=True)
    iota = lax.broadcasted_iota(jnp.int32, (_BT, E), 1)
    p1 = jnp.max(p, axis=1, keepdims=True)
    i1 = jnp.min(jnp.where(p == p1, iota, E), axis=1, keepdims=True)
    pm = jnp.where(iota == i1, -1.0, p)
    p2 = jnp.max(pm, axis=1, keepdims=True)
    i2 = jnp.min(jnp.where(pm == p2, iota, E), axis=1, keepdims=True)
    den = p1 + p2
    eidx_ref[...] = jnp.concatenate([i1, i2], axis=1)
    gate_ref[...] = jnp.concatenate([p1 / den, p2 / den], axis=1)


def _route(x, mu, cov_inv):
    return pl.pallas_call(
        _route_body,
        grid=(T // _BT,),
        in_specs=[
            pl.BlockSpec((_BT, D), lambda i: (i, 0)),
            pl.BlockSpec((D, D), lambda i: (0, 0)),
            pl.BlockSpec((E, D), lambda i: (0, 0)),
        ],
        out_specs=[
            pl.BlockSpec((_BT, K), lambda i: (i, 0)),
            pl.BlockSpec((_BT, K), lambda i: (i, 0)),
        ],
        out_shape=[
            jax.ShapeDtypeStruct((T, K), jnp.int32),
            jax.ShapeDtypeStruct((T, K), jnp.float32),
        ],
    )(x, cov_inv, mu)


def _disp_body(e_ref, g_ref, slot_ref, scale_ref):
    ef = e_ref[...]
    eiota = lax.broadcasted_iota(jnp.int32, (_NB, E, _PB), 1)
    oh = (ef[:, None, :] == eiota).astype(jnp.float32)
    ii = lax.broadcasted_iota(jnp.int32, (_PB, _PB), 0)
    jj = lax.broadcasted_iota(jnp.int32, (_PB, _PB), 1)
    tri = (jj <= ii).astype(jnp.float32)
    within = lax.dot_general(oh, tri, (((2,), (1,)), ((), ())),
                             preferred_element_type=jnp.float32)
    totals = jnp.sum(oh, axis=2)
    bi = lax.broadcasted_iota(jnp.int32, (_NB, _NB), 0)
    bj = lax.broadcasted_iota(jnp.int32, (_NB, _NB), 1)
    triex = (bj < bi).astype(jnp.float32)
    offs = lax.dot_general(triex, totals, (((1,), (0,)), ((), ())))
    pos = jnp.sum(oh * (within + offs[:, :, None]), axis=1) - 1.0
    posi = pos.astype(jnp.int32)
    keep = posi < CAP
    posc = jnp.minimum(posi, CAP - 1)
    slot_ref[...] = jnp.where(keep, ef * CAPP + posc, ef * CAPP + CAP)
    scale_ref[...] = jnp.where(keep, g_ref[...], 0.0)


def _dispatch(e2d, g2d):
    return pl.pallas_call(
        _disp_body,
        out_shape=[
            jax.ShapeDtypeStruct((_NB, _PB), jnp.int32),
            jax.ShapeDtypeStruct((_NB, _PB), jnp.float32),
        ],
    )(e2d, g2d)


_ST = T // _NW


def _scatterx_body(x_hbm, s0_hbm, s1_hbm, xg_hbm, slab_v, i0_v, i1_v, sem):
    wid = lax.axis_index("s") * _NC + lax.axis_index("c")
    tbase = wid * _ST
    ca = pltpu.async_copy(x_hbm.at[pl.ds(tbase, _ST)], slab_v, sem)
    cb = pltpu.async_copy(s0_hbm.at[pl.ds(tbase, _ST)], i0_v, sem)
    cc = pltpu.async_copy(s1_hbm.at[pl.ds(tbase, _ST)], i1_v, sem)
    ca.wait()
    cb.wait()
    cc.wait()
    c0 = pltpu.async_copy(slab_v, xg_hbm.at[i0_v], sem)
    c1 = pltpu.async_copy(slab_v, xg_hbm.at[i1_v], sem)
    c0.wait()
    c1.wait()


def _scatter_x(x, slot0, slot1):
    mesh = plsc.VectorSubcoreMesh(core_axis_name="c", subcore_axis_name="s")
    return pl.kernel(
        _scatterx_body,
        mesh=mesh,
        out_type=jax.ShapeDtypeStruct((_NSLOT, D), jnp.float32),
        scratch_types=[
            pltpu.VMEM((_ST, D), jnp.float32),
            pltpu.VMEM((_ST,), jnp.int32),
            pltpu.VMEM((_ST,), jnp.int32),
            pltpu.SemaphoreType.DMA,
        ],
    )(x, slot0, slot1)


def _ffn_body(xg_ref, w1_ref, b1_ref, w2_ref, b2_ref, out_ref):
    xg = xg_ref[0]
    h = jnp.dot(xg, w1_ref[0], preferred_element_type=jnp.float32)
    h = jnp.maximum(h + b1_ref[0], 0.0)
    o = jnp.dot(h, w2_ref[0], preferred_element_type=jnp.float32)
    out_ref[0] = o + b2_ref[0]


def _ffn(xg, W1, b1, W2, b2):
    return pl.pallas_call(
        _ffn_body,
        grid=(E,),
        in_specs=[
            pl.BlockSpec((1, CAPP, D), lambda e: (e, 0, 0)),
            pl.BlockSpec((1, D, F), lambda e: (e, 0, 0)),
            pl.BlockSpec((1, 1, F), lambda e: (e, 0, 0)),
            pl.BlockSpec((1, F, D), lambda e: (e, 0, 0)),
            pl.BlockSpec((1, 1, D), lambda e: (e, 0, 0)),
        ],
        out_specs=pl.BlockSpec((1, CAPP, D), lambda e: (e, 0, 0)),
        out_shape=jax.ShapeDtypeStruct((E, CAPP, D), jnp.float32),
    )(xg, W1, b1.reshape(E, 1, F), W2, b2.reshape(E, 1, D))


_CR = _NP // _NW
_CCH = 32


_GH = _ST // 2


def _gathero_body(oute_hbm, f0_hbm, f1_hbm, g0_hbm, g1_hbm,
                  b0, b1, b2, i0, i1, i2, i3, sem, isem):
    wid = lax.axis_index("s") * _NC + lax.axis_index("c")
    base = wid * _ST
    bufs = (b0, b1, b2)
    idxs = (i0, i1, i2, i3)
    chunks = ((f0_hbm, g0_hbm, 0), (f0_hbm, g0_hbm, _GH),
              (f1_hbm, g1_hbm, 0), (f1_hbm, g1_hbm, _GH))
    pend = {}
    for r in range(2):
        src, _, off = chunks[r]
        pltpu.async_copy(src.at[pl.ds(base + off, _GH)], idxs[r], isem).wait()
        pend[r] = pltpu.async_copy(oute_hbm.at[idxs[r]], bufs[r % 3], sem)
    for r in range(4):
        pend[r].wait()
        if r + 2 < 4:
            src, _, off = chunks[r + 2]
            pltpu.async_copy(
                src.at[pl.ds(base + off, _GH)], idxs[r + 2], isem).wait()
            pend[r + 2] = pltpu.async_copy(
                oute_hbm.at[idxs[r + 2]], bufs[(r + 2) % 3], sem)
        _, dst, off = chunks[r]
        pltpu.sync_copy(bufs[r % 3], dst.at[pl.ds(base + off, _GH)])


def _gather_o(oute, fsrc0, fsrc1):
    mesh = plsc.VectorSubcoreMesh(core_axis_name="c", subcore_axis_name="s")
    return pl.kernel(
        _gathero_body,
        mesh=mesh,
        out_type=[
            jax.ShapeDtypeStruct((T, D), jnp.float32),
            jax.ShapeDtypeStruct((T, D), jnp.float32),
        ],
        scratch_types=[
            pltpu.VMEM((_GH, D), jnp.float32),
            pltpu.VMEM((_GH, D), jnp.float32),
            pltpu.VMEM((_GH, D), jnp.float32),
            pltpu.VMEM((_GH,), jnp.int32),
            pltpu.VMEM((_GH,), jnp.int32),
            pltpu.VMEM((_GH,), jnp.int32),
            pltpu.VMEM((_GH,), jnp.int32),
            pltpu.SemaphoreType.DMA,
            pltpu.SemaphoreType.DMA,
        ],
    )(oute, fsrc0, fsrc1)


def _combine_body(g0_ref, g1_ref, s_ref, y_ref):
    u0 = g0_ref[...]
    u1 = g1_ref[...]
    s = s_ref[...]
    y_ref[...] = u0 * s[:, 0:1] + u1 * s[:, 1:2]


def _combine(g0, g1, scale):
    return pl.pallas_call(
        _combine_body,
        grid=(T // _BT,),
        in_specs=[
            pl.BlockSpec((_BT, D), lambda i: (i, 0)),
            pl.BlockSpec((_BT, D), lambda i: (i, 0)),
            pl.BlockSpec((_BT, K), lambda i: (i, 0)),
        ],
        out_specs=pl.BlockSpec((_BT, D), lambda i: (i, 0)),
        out_shape=jax.ShapeDtypeStruct((T, D), jnp.float32),
    )(g0, g1, scale)


def kernel(x, mu, cov_inv, W1, b1, W2, b2):
    eidx, gates = _route(x, mu, cov_inv)
    slot2d, scale2d = _dispatch(eidx.reshape(_NB, _PB), gates.reshape(_NB, _PB))
    slotk = slot2d.reshape(T, K)
    xg = _scatter_x(x, slotk[:, 0], slotk[:, 1])
    oute = _ffn(xg.reshape(E, CAPP, D), W1, b1, W2, b2)
    g0, g1 = _gather_o(oute.reshape(_NSLOT, D), slotk[:, 0], slotk[:, 1])
    y = _combine(g0, g1, scale2d.reshape(T, K))
    return y
